# baseline (speedup 1.0000x reference)
import functools

import jax
import jax.numpy as jnp
from jax.experimental import pallas as pl
from jax.experimental.pallas import tpu as pltpu


def _lane_tree_sum(xb, hw):
    if hw % 128 == 0 and hw > 128:
        parts = [xb[:, :, j * 128:(j + 1) * 128] for j in range(hw // 128)]
        while len(parts) > 1:
            nxt = [parts[i] + parts[i + 1] for i in range(0, len(parts) - 1, 2)]
            if len(parts) % 2:
                nxt.append(parts[-1])
            parts = nxt
        return jnp.sum(parts[0], axis=-1)
    return jnp.sum(xb, axis=-1)


def _se_manual_body(x_hbm, w1t_ref, b1_ref, w2t_ref, b2_ref, o_hbm,
                    buf, in_sems, out_sems, *, n_chunks, nbuf, bt, inv_hw):
    hw = buf.shape[-1]

    def cin(i, s):
        return pltpu.make_async_copy(
            x_hbm.at[pl.ds(i * bt, bt)], buf.at[s], in_sems.at[s])

    def cout(i, s):
        return pltpu.make_async_copy(
            buf.at[s], o_hbm.at[pl.ds(i * bt, bt)], out_sems.at[s])

    for k in range(min(nbuf, n_chunks)):
        cin(k, k).start()

    for i in range(n_chunks):
        s = i % nbuf
        cin(i, s).wait()
        xb = buf[s]
        y = _lane_tree_sum(xb, hw) * inv_hw
        h = jnp.dot(y, w1t_ref[...], preferred_element_type=jnp.float32)
        h = jnp.maximum(h + b1_ref[...], 0.0)
        z = jnp.dot(h, w2t_ref[...], preferred_element_type=jnp.float32)
        z = z + b2_ref[...]
        g = 0.5 * (1.0 + jnp.tanh(0.5 * z))
        buf[s] = xb * g[:, :, None]
        cout(i, s).start()
        nxt = i + nbuf
        if nxt < n_chunks:
            cout(i, s).wait()
            cin(nxt, s).start()

    for i in range(max(0, n_chunks - nbuf), n_chunks):
        cout(i, i % nbuf).wait()


def _pick_bt(B, slab_bytes, target_bytes=4 << 20):
    best = 1
    for d in range(1, B + 1):
        if B % d == 0 and d * slab_bytes <= target_bytes:
            best = d
    return best


def kernel(x, w1, b1, w2, b2):
    B, C, H, W = x.shape
    HW = H * W
    Ch = w1.shape[0]
    itemsize = jnp.dtype(x.dtype).itemsize

    xf = x.astype(jnp.float32).reshape(B, C, HW)
    w1t = w1.T.astype(jnp.float32)
    b1r = b1.reshape(1, Ch).astype(jnp.float32)
    w2t = w2.T.astype(jnp.float32)
    b2r = b2.reshape(1, C).astype(jnp.float32)

    slab = C * HW * itemsize
    bt = _pick_bt(B, slab, target_bytes=2 << 20)
    n_chunks = B // bt
    nbuf = min(8, n_chunks)

    vspec = pl.BlockSpec(memory_space=pltpu.MemorySpace.VMEM)
    cost = pl.CostEstimate(
        flops=int(2 * B * C * HW + 4 * B * C * Ch),
        transcendentals=int(B * C),
        bytes_accessed=int(2 * B * C * HW * itemsize),
    )
    out = pl.pallas_call(
        functools.partial(_se_manual_body, n_chunks=n_chunks, nbuf=nbuf,
                          bt=bt, inv_hw=1.0 / HW),
        out_shape=jax.ShapeDtypeStruct((B, C, HW), jnp.float32),
        in_specs=[pl.BlockSpec(memory_space=pl.ANY),
                  vspec, vspec, vspec, vspec],
        out_specs=pl.BlockSpec(memory_space=pl.ANY),
        scratch_shapes=[
            pltpu.VMEM((nbuf, bt, C, HW), jnp.float32),
            pltpu.SemaphoreType.DMA((nbuf,)),
            pltpu.SemaphoreType.DMA((nbuf,)),
        ],
        compiler_params=pltpu.CompilerParams(
            vmem_limit_bytes=int(min(56 << 20,
                                     nbuf * bt * slab + (8 << 20))),
        ),
        cost_estimate=cost,
    )(xf, w1t, b1r, w2t, b2r)
    return out.reshape(B, C, H, W).astype(x.dtype)


# speedup vs baseline: 1.0389x; 1.0389x over previous
import functools

import jax
import jax.numpy as jnp
from jax.experimental import pallas as pl
from jax.experimental.pallas import tpu as pltpu


def _lane_tree_sum(xb, hw):
    if hw % 128 == 0 and hw > 128:
        parts = [xb[:, :, j * 128:(j + 1) * 128] for j in range(hw // 128)]
        while len(parts) > 1:
            nxt = [parts[i] + parts[i + 1] for i in range(0, len(parts) - 1, 2)]
            if len(parts) % 2:
                nxt.append(parts[-1])
            parts = nxt
        return jnp.sum(parts[0], axis=-1)
    return jnp.sum(xb, axis=-1)


def _se_manual_body(x_hbm, w1t_ref, b1_ref, w2t_ref, b2_ref, o_hbm,
                    buf, in_sems, out_sems, *, n_chunks, nbuf, bt, inv_hw):
    hw = buf.shape[-1]

    def cin(i, s):
        return pltpu.make_async_copy(
            x_hbm.at[pl.ds(i * bt, bt)], buf.at[s], in_sems.at[s])

    def cout(i, s):
        return pltpu.make_async_copy(
            buf.at[s], o_hbm.at[pl.ds(i * bt, bt)], out_sems.at[s])

    for k in range(min(nbuf, n_chunks)):
        cin(k, k).start()

    for i in range(n_chunks):
        s = i % nbuf
        cin(i, s).wait()
        xb = buf[s]
        y = _lane_tree_sum(xb, hw) * inv_hw
        h = jnp.dot(y, w1t_ref[...], preferred_element_type=jnp.float32)
        h = jnp.maximum(h + b1_ref[...], 0.0)
        z = jnp.dot(h, w2t_ref[...], preferred_element_type=jnp.float32)
        z = z + b2_ref[...]
        g = 0.5 * (1.0 + jnp.tanh(0.5 * z))
        buf[s] = xb * g[:, :, None]
        cout(i, s).start()
        nxt = i + nbuf
        if nxt < n_chunks:
            cout(i, s).wait()
            cin(nxt, s).start()

    for i in range(max(0, n_chunks - nbuf), n_chunks):
        cout(i, i % nbuf).wait()


def _pick_bt(B, slab_bytes, target_bytes=4 << 20):
    best = 1
    for d in range(1, B + 1):
        if B % d == 0 and d * slab_bytes <= target_bytes:
            best = d
    return best


def kernel(x, w1, b1, w2, b2):
    B, C, H, W = x.shape
    HW = H * W
    Ch = w1.shape[0]
    itemsize = jnp.dtype(x.dtype).itemsize

    xf = x.astype(jnp.float32).reshape(B, C, HW)
    w1t = w1.T.astype(jnp.float32)
    b1r = b1.reshape(1, Ch).astype(jnp.float32)
    w2t = w2.T.astype(jnp.float32)
    b2r = b2.reshape(1, C).astype(jnp.float32)

    slab = C * HW * itemsize
    bt = _pick_bt(B, slab, target_bytes=8 << 20)
    n_chunks = B // bt
    nbuf = min(4, n_chunks)

    vspec = pl.BlockSpec(memory_space=pltpu.MemorySpace.VMEM)
    cost = pl.CostEstimate(
        flops=int(2 * B * C * HW + 4 * B * C * Ch),
        transcendentals=int(B * C),
        bytes_accessed=int(2 * B * C * HW * itemsize),
    )
    out = pl.pallas_call(
        functools.partial(_se_manual_body, n_chunks=n_chunks, nbuf=nbuf,
                          bt=bt, inv_hw=1.0 / HW),
        out_shape=jax.ShapeDtypeStruct((B, C, HW), jnp.float32),
        in_specs=[pl.BlockSpec(memory_space=pl.ANY),
                  vspec, vspec, vspec, vspec],
        out_specs=pl.BlockSpec(memory_space=pl.ANY),
        scratch_shapes=[
            pltpu.VMEM((nbuf, bt, C, HW), jnp.float32),
            pltpu.SemaphoreType.DMA((nbuf,)),
            pltpu.SemaphoreType.DMA((nbuf,)),
        ],
        compiler_params=pltpu.CompilerParams(
            vmem_limit_bytes=int(min(56 << 20,
                                     nbuf * bt * slab + (8 << 20))),
        ),
        cost_estimate=cost,
    )(xf, w1t, b1r, w2t, b2r)
    return out.reshape(B, C, H, W).astype(x.dtype)


# speedup vs baseline: 1.0420x; 1.0029x over previous
import functools

import jax
import jax.numpy as jnp
from jax.experimental import pallas as pl
from jax.experimental.pallas import tpu as pltpu


def _lane_tree_sum(xb, hw):
    if hw % 128 == 0 and hw > 128:
        parts = [xb[:, :, j * 128:(j + 1) * 128] for j in range(hw // 128)]
        while len(parts) > 1:
            nxt = [parts[i] + parts[i + 1] for i in range(0, len(parts) - 1, 2)]
            if len(parts) % 2:
                nxt.append(parts[-1])
            parts = nxt
        return jnp.sum(parts[0], axis=-1)
    return jnp.sum(xb, axis=-1)


def _se_manual_body(x_hbm, w1t_ref, b1_ref, w2t_ref, b2_ref, o_hbm,
                    buf, in_sems, out_sems, *, n_chunks, nbuf, bt, inv_hw):
    hw = buf.shape[-1]

    def cin(i, s):
        return pltpu.make_async_copy(
            x_hbm.at[pl.ds(i * bt, bt)], buf.at[s], in_sems.at[s])

    def cout(i, s):
        return pltpu.make_async_copy(
            buf.at[s], o_hbm.at[pl.ds(i * bt, bt)], out_sems.at[s])

    for k in range(min(nbuf, n_chunks)):
        cin(k, k).start()

    for i in range(n_chunks):
        s = i % nbuf
        cin(i, s).wait()
        xb = buf[s]
        y = _lane_tree_sum(xb, hw) * inv_hw
        h = jnp.dot(y, w1t_ref[...], preferred_element_type=jnp.float32)
        h = jnp.maximum(h + b1_ref[...], 0.0)
        z = jnp.dot(h, w2t_ref[...], preferred_element_type=jnp.float32)
        z = z + b2_ref[...]
        g = 0.5 * (1.0 + jnp.tanh(0.5 * z))
        buf[s] = xb * g[:, :, None]
        cout(i, s).start()
        nxt = i + nbuf
        if nxt < n_chunks:
            cout(i, s).wait()
            cin(nxt, s).start()

    for i in range(max(0, n_chunks - nbuf), n_chunks):
        cout(i, i % nbuf).wait()


def _pick_bt(B, slab_bytes, target_bytes=4 << 20):
    best = 1
    for d in range(1, B + 1):
        if B % d == 0 and d * slab_bytes <= target_bytes:
            best = d
    return best


def kernel(x, w1, b1, w2, b2):
    B, C, H, W = x.shape
    HW = H * W
    Ch = w1.shape[0]
    itemsize = jnp.dtype(x.dtype).itemsize

    xf = x.astype(jnp.float32).reshape(B, C, HW)
    w1t = w1.T.astype(jnp.float32)
    b1r = b1.reshape(1, Ch).astype(jnp.float32)
    w2t = w2.T.astype(jnp.float32)
    b2r = b2.reshape(1, C).astype(jnp.float32)

    slab = C * HW * itemsize
    bt = _pick_bt(B, slab, target_bytes=16 << 20)
    n_chunks = B // bt
    nbuf = min(2, n_chunks)

    vspec = pl.BlockSpec(memory_space=pltpu.MemorySpace.VMEM)
    cost = pl.CostEstimate(
        flops=int(2 * B * C * HW + 4 * B * C * Ch),
        transcendentals=int(B * C),
        bytes_accessed=int(2 * B * C * HW * itemsize),
    )
    out = pl.pallas_call(
        functools.partial(_se_manual_body, n_chunks=n_chunks, nbuf=nbuf,
                          bt=bt, inv_hw=1.0 / HW),
        out_shape=jax.ShapeDtypeStruct((B, C, HW), jnp.float32),
        in_specs=[pl.BlockSpec(memory_space=pl.ANY),
                  vspec, vspec, vspec, vspec],
        out_specs=pl.BlockSpec(memory_space=pl.ANY),
        scratch_shapes=[
            pltpu.VMEM((nbuf, bt, C, HW), jnp.float32),
            pltpu.SemaphoreType.DMA((nbuf,)),
            pltpu.SemaphoreType.DMA((nbuf,)),
        ],
        compiler_params=pltpu.CompilerParams(
            vmem_limit_bytes=int(min(56 << 20,
                                     nbuf * bt * slab + (8 << 20))),
        ),
        cost_estimate=cost,
    )(xf, w1t, b1r, w2t, b2r)
    return out.reshape(B, C, H, W).astype(x.dtype)
